# baseline (speedup 1.0000x reference)
import functools

import jax
import jax.numpy as jnp
from jax import lax
from jax.experimental import pallas as pl
from jax.experimental.pallas import tpu as pltpu
from jax.experimental.pallas import tpu_sc as plsc

_NC, _NS = 2, 16
_NW = _NC * _NS
_BB = 128
_NGB = 3
_NPB = 2


@functools.lru_cache(maxsize=None)
def _build(B, S, D):
    NBB = B // _BB
    units = S * NBB
    upt = units // _NW
    G = D // 8
    mesh = plsc.VectorSubcoreMesh(core_axis_name="c", subcore_axis_name="s")

    @functools.partial(
        pl.kernel,
        out_type=jax.ShapeDtypeStruct((S, G, NBB, 8, _BB), jnp.float32),
        mesh=mesh,
        scratch_types=[
            pltpu.VMEM((upt, _BB), jnp.int32),
            pltpu.VMEM((upt, _BB), jnp.int32),
            pltpu.VMEM((upt, _BB), jnp.int32),
            pltpu.VMEM((S * D,), jnp.float32),
            pltpu.VMEM((_NGB, _BB, 2 * D), jnp.float32),
            pltpu.VMEM((_NPB, G, 8, _BB), jnp.float32),
        ] + [pltpu.SemaphoreType.DMA] * (_NGB + _NPB),
        compiler_params=pltpu.CompilerParams(
            use_tc_tiling_on_sc=False, needs_layout_passes=False),
    )
    def k(xt_hbm, tab_hbm, pe_hbm, out_hbm, idxr, idxp, parb, pe_v, bufs,
          planes, *sems):
        gsems = sems[:_NGB]
        osems = sems[_NGB:]
        wid = lax.axis_index("s") * _NC + lax.axis_index("c")
        u0 = wid * upt
        pltpu.sync_copy(xt_hbm.at[pl.ds(u0, upt)], idxr)
        pltpu.sync_copy(pe_hbm, pe_v)

        def prep_row(r, carry):
            for g in range(_BB // 16):
                sl = pl.ds(g * 16, 16)
                v = idxr[r, sl]
                idxp[r, sl] = lax.shift_right_logical(v, 1)
                parb[r, sl] = (v & 1) * D
            return carry

        lax.fori_loop(0, upt, prep_row, 0)

        lane = lax.iota(jnp.int32, 16)
        rows16 = [lane + bb * 16 for bb in range(_BB // 16)]

        def fire_gather(u, gb):
            pltpu.make_async_copy(
                tab_hbm.at[idxp.at[u]], bufs.at[gb], gsems[gb]).start()

        def wait_gather(gb):
            pltpu.make_async_copy(
                tab_hbm.at[idxp.at[0]], bufs.at[gb], gsems[gb]).wait()

        def fire_plane(u, pb):
            gu = u0 + u
            s = gu // NBB
            c = gu - s * NBB
            pltpu.make_async_copy(
                planes.at[pb], out_hbm.at[s, :, c], osems[pb]).start()

        def wait_plane(pb):
            pltpu.make_async_copy(
                planes.at[pb], out_hbm.at[0, :, 0], osems[pb]).wait()

        def do_unit(u, gb, pb, first_pb, last_fire):
            wait_gather(gb)
            if not first_pb:
                wait_plane(pb)
            gu = u0 + u
            s = gu // NBB
            buf = bufs.at[gb]

            def d_body(d, carry):
                dg = d // 8
                dd = d - dg * 8
                pvec = plsc.load_gather(
                    pe_v, [jnp.broadcast_to(s * D + d, (16,))])
                for bb in range(_BB // 16):
                    col = parb[u, pl.ds(bb * 16, 16)] + d
                    vals = plsc.load_gather(buf, [rows16[bb], col])
                    planes[pb, dg, dd, pl.ds(bb * 16, 16)] = vals + pvec
                return carry

            lax.fori_loop(0, D, d_body, 0)
            fire_plane(u, pb)
            if not last_fire:
                fire_gather(u + _NPB, (gb + _NPB) % _NGB)

        fire_gather(0, 0)
        fire_gather(1, 1)
        do_unit(0, 0, 0, first_pb=True, last_fire=False)
        do_unit(1, 1, 1, first_pb=True, last_fire=False)

        def block(blk, carry):
            for j in range(6):
                u = 2 + blk * 6 + j
                do_unit(u, (2 + j) % _NGB, j % _NPB,
                        first_pb=False, last_fire=False)
            return carry

        n_blocks = (upt - 2) // 6 - 1
        lax.fori_loop(0, n_blocks, block, 0)

        base = 2 + n_blocks * 6
        for j in range(upt - base):
            u = base + j
            do_unit(u, u % _NGB, u % _NPB,
                    first_pb=False, last_fire=(u + _NPB >= upt))
        for pb in range(_NPB):
            wait_plane(pb)

    return k


def kernel(x, embedding_matrix, positional_encodings):
    B, S = x.shape
    V, D = embedding_matrix.shape
    xt = x.T.reshape(S * B // _BB, _BB).astype(jnp.int32)
    tab2 = embedding_matrix.reshape(V // 2, 2 * D)
    pe = positional_encodings[:S].reshape(S * D)
    o5 = _build(B, S, D)(xt, tab2, pe)
    return o5.transpose(2, 4, 0, 1, 3).reshape(B, S, D)


# speedup vs baseline: 1.1691x; 1.1691x over previous
import functools

import jax
import jax.numpy as jnp
from jax import lax
from jax.experimental import pallas as pl
from jax.experimental.pallas import tpu as pltpu
from jax.experimental.pallas import tpu_sc as plsc

_NC, _NS = 2, 16
_NW = _NC * _NS
_BB = 128
_NGB = 4
_NPB = 2


@functools.lru_cache(maxsize=None)
def _build(B, S, D):
    NBB = B // _BB
    units = S * NBB
    upt = units // _NW
    G = D // 8
    mesh = plsc.VectorSubcoreMesh(core_axis_name="c", subcore_axis_name="s")

    @functools.partial(
        pl.kernel,
        out_type=jax.ShapeDtypeStruct((S, G, NBB, 8, _BB), jnp.float32),
        mesh=mesh,
        scratch_types=[
            pltpu.VMEM((upt, _BB), jnp.int32),
            pltpu.VMEM((upt, _BB), jnp.int32),
            pltpu.VMEM((upt, _BB), jnp.int32),
            pltpu.VMEM((S * D,), jnp.float32),
            pltpu.VMEM((_NGB, _BB, 2 * D), jnp.float32),
            pltpu.VMEM((_NPB, G, 8, _BB), jnp.float32),
        ] + [pltpu.SemaphoreType.DMA] * (_NGB + _NPB),
        compiler_params=pltpu.CompilerParams(
            use_tc_tiling_on_sc=False, needs_layout_passes=False),
    )
    def k(xt_hbm, tab_hbm, pe_hbm, out_hbm, idxr, idxp, parb, pe_v, bufs,
          planes, *sems):
        gsems = sems[:_NGB]
        osems = sems[_NGB:]
        wid = lax.axis_index("s") * _NC + lax.axis_index("c")
        u0 = wid * upt
        pltpu.sync_copy(xt_hbm.at[pl.ds(u0, upt)], idxr)
        pltpu.sync_copy(pe_hbm, pe_v)

        def prep_row(r, carry):
            for g in range(_BB // 16):
                sl = pl.ds(g * 16, 16)
                v = idxr[r, sl]
                idxp[r, sl] = lax.shift_right_logical(v, 1)
                parb[r, sl] = (v & 1) * D
            return carry

        lax.fori_loop(0, upt, prep_row, 0)

        lane = lax.iota(jnp.int32, 16)
        rows16 = [lane + bb * 16 for bb in range(_BB // 16)]

        def fire_gather(u, gb):
            pltpu.make_async_copy(
                tab_hbm.at[idxp.at[u]], bufs.at[gb], gsems[gb]).start()

        def wait_gather(gb):
            pltpu.make_async_copy(
                tab_hbm.at[idxp.at[0]], bufs.at[gb], gsems[gb]).wait()

        def fire_plane(u, pb):
            gu = u0 + u
            s = gu // NBB
            c = gu - s * NBB
            pltpu.make_async_copy(
                planes.at[pb], out_hbm.at[s, :, c], osems[pb]).start()

        def wait_plane(pb):
            pltpu.make_async_copy(
                planes.at[pb], out_hbm.at[0, :, 0], osems[pb]).wait()

        def do_unit(u, gb, pb, first_pb, last_fire):
            wait_gather(gb)
            if not first_pb:
                wait_plane(pb)
            gu = u0 + u
            s = lax.shift_right_logical(gu, 3)
            buf = bufs.at[gb]
            pars = tuple(
                parb[u, pl.ds(bb * 16, 16)] for bb in range(_BB // 16))

            def dg_body(dg, carry):
                sdg = s * D + dg * 8
                for dd in range(8):
                    d = dg * 8 + dd
                    dvec = jnp.broadcast_to(d, (16,))
                    pvec = plsc.load_gather(
                        pe_v, [jnp.broadcast_to(sdg + dd, (16,))])
                    for bb in range(_BB // 16):
                        vals = plsc.load_gather(
                            buf, [rows16[bb], carry[bb] + dvec])
                        planes[pb, dg, dd, pl.ds(bb * 16, 16)] = vals + pvec
                return carry

            lax.fori_loop(0, 8, dg_body, pars)
            fire_plane(u, pb)
            if not last_fire:
                fire_gather(u + _NPB, (gb + _NPB) % _NGB)

        fire_gather(0, 0)
        fire_gather(1, 1)
        do_unit(0, 0, 0, first_pb=True, last_fire=False)
        do_unit(1, 1, 1, first_pb=True, last_fire=False)

        def block(blk, carry):
            for j in range(_NGB):
                u = 2 + blk * _NGB + j
                do_unit(u, (2 + j) % _NGB, j % _NPB,
                        first_pb=False, last_fire=False)
            return carry

        n_blocks = (upt - 2) // _NGB - 1
        lax.fori_loop(0, n_blocks, block, 0)

        base = 2 + n_blocks * _NGB
        for j in range(upt - base):
            u = base + j
            do_unit(u, u % _NGB, u % _NPB,
                    first_pb=False, last_fire=(u + _NPB >= upt))
        for pb in range(_NPB):
            wait_plane(pb)

    return k


def kernel(x, embedding_matrix, positional_encodings):
    B, S = x.shape
    V, D = embedding_matrix.shape
    xt = x.T.reshape(S * B // _BB, _BB).astype(jnp.int32)
    tab2 = embedding_matrix.reshape(V // 2, 2 * D)
    pe = positional_encodings[:S].reshape(S * D)
    o5 = _build(B, S, D)(xt, tab2, pe)
    return o5.transpose(2, 4, 0, 1, 3).reshape(B, S, D)


# speedup vs baseline: 1.2867x; 1.1006x over previous
import functools

import jax
import jax.numpy as jnp
from jax import lax
from jax.experimental import pallas as pl
from jax.experimental.pallas import tpu as pltpu
from jax.experimental.pallas import tpu_sc as plsc

_NC, _NS = 2, 16
_NW = _NC * _NS
_BB = 128
_NGB = 4
_NPB = 2


@functools.lru_cache(maxsize=None)
def _build(B, S, D):
    NBB = B // _BB
    units = S * NBB
    upt = units // _NW
    G = D // 8
    mesh = plsc.VectorSubcoreMesh(core_axis_name="c", subcore_axis_name="s")

    @functools.partial(
        pl.kernel,
        out_type=jax.ShapeDtypeStruct((S, G, NBB, 8, _BB), jnp.float32),
        mesh=mesh,
        scratch_types=[
            pltpu.VMEM((upt, _BB), jnp.int32),
            pltpu.VMEM((upt, _BB), jnp.int32),
            pltpu.VMEM((upt, _BB), jnp.int32),
            pltpu.VMEM((S * D,), jnp.float32),
            pltpu.VMEM((_NGB, _BB, 2 * D), jnp.float32),
            pltpu.VMEM((_NPB, G, 8, _BB), jnp.float32),
        ] + [pltpu.SemaphoreType.DMA] * (_NGB + _NPB),
        compiler_params=pltpu.CompilerParams(
            use_tc_tiling_on_sc=False, needs_layout_passes=False),
    )
    def k(xt_hbm, tab_hbm, pe_hbm, out_hbm, idxr, idxp, parb, pe_v, bufs,
          planes, *sems):
        gsems = sems[:_NGB]
        osems = sems[_NGB:]
        wid = lax.axis_index("s") * _NC + lax.axis_index("c")
        u0 = wid * upt
        pltpu.sync_copy(xt_hbm.at[pl.ds(u0, upt)], idxr)
        pltpu.sync_copy(pe_hbm, pe_v)

        def prep_row(r, carry):
            for g in range(_BB // 16):
                sl = pl.ds(g * 16, 16)
                v = idxr[r, sl]
                idxp[r, sl] = lax.shift_right_logical(v, 1)
                parb[r, sl] = (v & 1) * D
            return carry

        lax.fori_loop(0, upt, prep_row, 0)

        lane = lax.iota(jnp.int32, 16)
        rows16 = [lane + bb * 16 for bb in range(_BB // 16)]

        def fire_gather(u, gb):
            pltpu.make_async_copy(
                tab_hbm.at[idxp.at[u]], bufs.at[gb], gsems[gb]).start()

        def wait_gather(gb):
            pltpu.make_async_copy(
                tab_hbm.at[idxp.at[0]], bufs.at[gb], gsems[gb]).wait()

        def fire_plane(u, pb):
            gu = u0 + u
            s = gu // NBB
            c = gu - s * NBB
            pltpu.make_async_copy(
                planes.at[pb], out_hbm.at[s, :, c], osems[pb]).start()

        def wait_plane(pb):
            pltpu.make_async_copy(
                planes.at[pb], out_hbm.at[0, :, 0], osems[pb]).wait()

        def do_unit(u, gb, pb, first_pb, last_fire):
            wait_gather(gb)
            if not first_pb:
                wait_plane(pb)
            gu = u0 + u
            s = lax.shift_right_logical(gu, 3)
            buf = bufs.at[gb]
            pars = tuple(
                parb[u, pl.ds(bb * 16, 16)] for bb in range(_BB // 16))
            sD = s * D

            def dg_body(dg, carry):
                base = sD + dg * 8
                for dd in range(8):
                    d = dg * 8 + dd
                    dvec = jnp.broadcast_to(d, (16,))
                    pvec = plsc.load_gather(
                        pe_v, [jnp.broadcast_to(base + dd, (16,))])
                    vals = [
                        plsc.load_gather(buf, [rows16[bb], pars[bb] + dvec])
                        for bb in range(_BB // 16)
                    ]
                    for bb in range(_BB // 16):
                        planes[pb, dg, dd, pl.ds(bb * 16, 16)] = (
                            vals[bb] + pvec)
                return carry

            lax.fori_loop(0, 8, dg_body, 0, unroll=2)
            fire_plane(u, pb)
            if not last_fire:
                fire_gather(u + _NPB, (gb + _NPB) % _NGB)

        fire_gather(0, 0)
        fire_gather(1, 1)
        do_unit(0, 0, 0, first_pb=True, last_fire=False)
        do_unit(1, 1, 1, first_pb=True, last_fire=False)

        def block(blk, carry):
            for j in range(_NGB):
                u = 2 + blk * _NGB + j
                do_unit(u, (2 + j) % _NGB, j % _NPB,
                        first_pb=False, last_fire=False)
            return carry

        n_blocks = (upt - 2) // _NGB - 1
        lax.fori_loop(0, n_blocks, block, 0)

        base = 2 + n_blocks * _NGB
        for j in range(upt - base):
            u = base + j
            do_unit(u, u % _NGB, u % _NPB,
                    first_pb=False, last_fire=(u + _NPB >= upt))
        for pb in range(_NPB):
            wait_plane(pb)

    return k


def kernel(x, embedding_matrix, positional_encodings):
    B, S = x.shape
    V, D = embedding_matrix.shape
    xt = x.T.reshape(S * B // _BB, _BB).astype(jnp.int32)
    tab2 = embedding_matrix.reshape(V // 2, 2 * D)
    pe = positional_encodings[:S].reshape(S * D)
    o5 = _build(B, S, D)(xt, tab2, pe)
    return o5.transpose(2, 4, 0, 1, 3).reshape(B, S, D)


# speedup vs baseline: 1.2867x; 1.0000x over previous
import functools

import jax
import jax.numpy as jnp
from jax import lax
from jax.experimental import pallas as pl
from jax.experimental.pallas import tpu as pltpu
from jax.experimental.pallas import tpu_sc as plsc

_NC, _NS = 2, 16
_NW = _NC * _NS
_BB = 128
_NGB = 2
_NPB = 2


@functools.lru_cache(maxsize=None)
def _build(B, S, D):
    NBB = B // _BB
    units = S * NBB
    upt = units // _NW
    G = D // 8
    mesh = plsc.VectorSubcoreMesh(core_axis_name="c", subcore_axis_name="s")

    @functools.partial(
        pl.kernel,
        out_type=jax.ShapeDtypeStruct((S, G, NBB, 8, _BB), jnp.float32),
        mesh=mesh,
        scratch_types=[
            pltpu.VMEM((upt, _BB), jnp.int32),
            pltpu.VMEM((upt, _BB), jnp.int32),
            pltpu.VMEM((upt, _BB), jnp.int32),
            pltpu.VMEM((S * D,), jnp.float32),
            pltpu.VMEM((_NGB, _BB, 2 * D), jnp.float32),
            pltpu.VMEM((_NPB, G, 8, _BB), jnp.float32),
        ] + [pltpu.SemaphoreType.DMA] * (_NGB + _NPB),
        compiler_params=pltpu.CompilerParams(
            use_tc_tiling_on_sc=False, needs_layout_passes=False),
    )
    def k(xt_hbm, tab_hbm, pe_hbm, out_hbm, idxr, idxp, parb, pe_v, bufs,
          planes, *sems):
        gsems = sems[:_NGB]
        osems = sems[_NGB:]
        wid = lax.axis_index("s") * _NC + lax.axis_index("c")
        u0 = wid * upt
        pltpu.sync_copy(xt_hbm.at[pl.ds(u0, upt)], idxr)
        pltpu.sync_copy(pe_hbm, pe_v)

        def prep_row(r, carry):
            for g in range(_BB // 16):
                sl = pl.ds(g * 16, 16)
                v = idxr[r, sl]
                idxp[r, sl] = lax.shift_right_logical(v, 1)
                parb[r, sl] = (v & 1) * D
            return carry

        lax.fori_loop(0, upt, prep_row, 0)

        lane = lax.iota(jnp.int32, 16)
        rows16 = [lane + bb * 16 for bb in range(_BB // 16)]

        def fire_gather(u, gb):
            pltpu.make_async_copy(
                tab_hbm.at[idxp.at[u]], bufs.at[gb], gsems[gb]).start()

        def wait_gather(gb):
            pltpu.make_async_copy(
                tab_hbm.at[idxp.at[0]], bufs.at[gb], gsems[gb]).wait()

        def fire_plane(u, pb):
            gu = u0 + u
            s = gu // NBB
            c = gu - s * NBB
            pltpu.make_async_copy(
                planes.at[pb], out_hbm.at[s, :, c], osems[pb]).start()

        def wait_plane(pb):
            pltpu.make_async_copy(
                planes.at[pb], out_hbm.at[0, :, 0], osems[pb]).wait()

        def do_unit(u, gb, pb, first_pb, last_fire):
            wait_gather(gb)
            if not first_pb:
                wait_plane(pb)
            gu = u0 + u
            s = lax.shift_right_logical(gu, 3)
            buf = bufs.at[gb]
            pars = tuple(
                parb[u, pl.ds(bb * 16, 16)] for bb in range(_BB // 16))
            sD = s * D

            def dg_body(dg, carry):
                base = sD + dg * 8
                for dd in range(8):
                    d = dg * 8 + dd
                    dvec = jnp.broadcast_to(d, (16,))
                    pvec = plsc.load_gather(
                        pe_v, [jnp.broadcast_to(base + dd, (16,))])
                    vals = [
                        plsc.load_gather(buf, [rows16[bb], pars[bb] + dvec])
                        for bb in range(_BB // 16)
                    ]
                    for bb in range(_BB // 16):
                        planes[pb, dg, dd, pl.ds(bb * 16, 16)] = (
                            vals[bb] + pvec)
                return carry

            lax.fori_loop(0, 8, dg_body, 0, unroll=4)
            fire_plane(u, pb)
            if not last_fire:
                fire_gather(u + _NPB, (gb + _NPB) % _NGB)

        fire_gather(0, 0)
        fire_gather(1, 1)
        do_unit(0, 0, 0, first_pb=True, last_fire=False)
        do_unit(1, 1, 1, first_pb=True, last_fire=False)

        def block(blk, carry):
            for j in range(_NGB):
                u = 2 + blk * _NGB + j
                do_unit(u, (2 + j) % _NGB, j % _NPB,
                        first_pb=False, last_fire=False)
            return carry

        n_blocks = (upt - 2) // _NGB - 1
        lax.fori_loop(0, n_blocks, block, 0)

        base = 2 + n_blocks * _NGB
        for j in range(upt - base):
            u = base + j
            do_unit(u, u % _NGB, u % _NPB,
                    first_pb=False, last_fire=(u + _NPB >= upt))
        for pb in range(_NPB):
            wait_plane(pb)

    return k


def kernel(x, embedding_matrix, positional_encodings):
    B, S = x.shape
    V, D = embedding_matrix.shape
    xt = x.T.reshape(S * B // _BB, _BB).astype(jnp.int32)
    tab2 = embedding_matrix.reshape(V // 2, 2 * D)
    pe = positional_encodings[:S].reshape(S * D)
    o5 = _build(B, S, D)(xt, tab2, pe)
    return o5.transpose(2, 4, 0, 1, 3).reshape(B, S, D)


# speedup vs baseline: 1.3302x; 1.0337x over previous
import functools

import jax
import jax.numpy as jnp
from jax import lax
from jax.experimental import pallas as pl
from jax.experimental.pallas import tpu as pltpu
from jax.experimental.pallas import tpu_sc as plsc

_NC, _NS = 2, 16
_NW = _NC * _NS
_CHUNK = 80
_NBUF = 5


@functools.lru_cache(maxsize=None)
def _build(n_chunks, S, D):
    assert (_CHUNK * _NBUF) % S == 0
    assert n_chunks % (_NW * _NBUF) == 0
    ncw = n_chunks // _NW
    ngroups = ncw // _NBUF
    n_rows = n_chunks * _CHUNK
    mesh = plsc.VectorSubcoreMesh(core_axis_name="c", subcore_axis_name="s")

    @functools.partial(
        pl.kernel,
        out_type=jax.ShapeDtypeStruct((n_rows, D), jnp.float32),
        mesh=mesh,
        scratch_types=[
            pltpu.VMEM((ncw, _CHUNK), jnp.int32),
            pltpu.VMEM((2 * S, D), jnp.float32),
            pltpu.VMEM((_NBUF, _CHUNK, D), jnp.float32),
        ] + [pltpu.SemaphoreType.DMA] * (2 * _NBUF),
        compiler_params=pltpu.CompilerParams(use_tc_tiling_on_sc=False),
    )
    def k(x_hbm, table_hbm, pe_hbm, out_hbm, idx_v, pe_v, rows_v, *sems):
        gsems = sems[:_NBUF]
        ssems = sems[_NBUF:]
        wid = lax.axis_index("s") * _NC + lax.axis_index("c")
        c0 = wid * ncw
        pltpu.sync_copy(x_hbm.at[pl.ds(c0, ncw)], idx_v)
        pltpu.sync_copy(pe_hbm, pe_v)

        def fire_gather(c_local, b):
            pltpu.make_async_copy(
                table_hbm.at[idx_v.at[c_local]], rows_v.at[b], gsems[b]
            ).start()

        def wait_gather(b):
            pltpu.make_async_copy(
                table_hbm.at[idx_v.at[0]], rows_v.at[b], gsems[b]
            ).wait()

        def fire_scatter(c_local, b):
            row0 = (c0 + c_local) * _CHUNK
            pltpu.make_async_copy(
                rows_v.at[b], out_hbm.at[pl.ds(row0, _CHUNK)], ssems[b]
            ).start()

        def wait_scatter(b):
            pltpu.make_async_copy(
                rows_v.at[b], out_hbm.at[pl.ds(0, _CHUNK)], ssems[b]
            ).wait()

        def add_pe(b):
            q = (_CHUNK * b) % S

            def row_body(r, carry):
                for gi in range(D // 16):
                    sl = pl.ds(gi * 16, 16)
                    rows_v[b, r, sl] = rows_v[b, r, sl] + pe_v[q + r, sl]
                return carry

            lax.fori_loop(0, _CHUNK, row_body, 0, unroll=2)

        def chunk_step(c_local, b, first, last):
            wait_gather(b)
            add_pe(b)
            nb = (b + _NBUF - 1) % _NBUF
            if not last:
                if not first:
                    wait_scatter(nb)
                fire_gather(c_local + (_NBUF - 1), nb)
            fire_scatter(c_local, b)

        for b in range(_NBUF - 1):
            fire_gather(b, b)
        for b in range(_NBUF):
            chunk_step(b, b, first=(b == 0), last=False)

        def group_body(g, carry):
            cbase = g * _NBUF
            for b in range(_NBUF):
                chunk_step(cbase + b, b, first=False, last=False)
            return carry

        lax.fori_loop(1, ngroups - 1, group_body, 0)

        cbase = (ngroups - 1) * _NBUF
        for b in range(_NBUF):
            chunk_step(cbase + b, b, first=False, last=(b != 0))
        for b in range(_NBUF):
            wait_scatter(b)

    return k


def kernel(x, embedding_matrix, positional_encodings):
    B, S = x.shape
    V, D = embedding_matrix.shape
    n_chunks = B * S // _CHUNK
    x2 = x.reshape(n_chunks, _CHUNK).astype(jnp.int32)
    pe = positional_encodings[:S]
    pe2 = jnp.concatenate([pe, pe], axis=0)
    out = _build(n_chunks, S, D)(x2, embedding_matrix, pe2)
    return out.reshape(B, S, D)
